# baseline (speedup 1.0000x reference)
import functools
import math

import jax
import jax.numpy as jnp
from jax import lax
from jax.experimental import pallas as pl
from jax.experimental.pallas import tpu as pltpu


def _cdiv(a, b):
    return -(-a // b)


def _round_up(x, m):
    return _cdiv(x, m) * m


def _gather_kernel(idx_ref, wblk_ref, out_ref, *, num_categories, pack):
    rows = out_ref.shape[0]
    c = num_categories

    idx = jnp.clip(idx_ref[...], 0, c - 1)
    iota_c = lax.broadcasted_iota(jnp.int32, (rows, c), 1)

    parts = [
        (iota_c == idx[:, g:g + 1]).astype(jnp.bfloat16)
        for g in range(pack)
    ]
    onehot = parts[0] if pack == 1 else jnp.concatenate(parts, axis=1)

    out_ref[...] = jax.lax.dot_general(
        onehot, wblk_ref[...],
        dimension_numbers=(((1,), (0,)), ((), ())),
        preferred_element_type=jnp.float32,
    )


def _pick_rows_tile(num_rows, cap=2048):
    if num_rows <= cap:
        return num_rows
    best = 0
    for t in range(cap, 7, -8):
        if num_rows % t == 0:
            best = t
            break
    if best:
        return best
    return min(_round_up(max(_cdiv(num_rows, 4), 1), 8), cap)


def kernel(category_indices, weight):
    C, D = weight.shape
    orig_shape = category_indices.shape

    idx = category_indices.reshape(-1).astype(jnp.int32)
    N = idx.shape[0]

    G = 128 // math.gcd(128, D)
    GD = G * D
    K = G * C

    w16 = weight.astype(jnp.bfloat16)
    w_blk = jnp.zeros((K, GD), dtype=jnp.bfloat16)
    for g in range(G):
        w_blk = w_blk.at[g * C:(g + 1) * C, g * D:(g + 1) * D].set(w16)

    Np = _round_up(N, G)
    if Np != N:
        idx = jnp.pad(idx, (0, Np - N))
    R = Np // G
    idx2 = idx.reshape(R, G)

    rows_tile = _pick_rows_tile(R)
    n_tiles = _cdiv(R, rows_tile)

    vmem_limit = int(min(
        2 * rows_tile * (G * 4 + GD * 4)
        + K * GD * 2
        + 2 * rows_tile * K * 2
        + (4 << 20), 100 << 20))

    out_packed = pl.pallas_call(
        functools.partial(_gather_kernel, num_categories=C, pack=G),
        out_shape=jax.ShapeDtypeStruct((R, GD), jnp.float32),
        grid=(n_tiles,),
        in_specs=[
            pl.BlockSpec((rows_tile, G), lambda i: (i, 0)),
            pl.BlockSpec((K, GD), lambda i: (0, 0)),
        ],
        out_specs=pl.BlockSpec((rows_tile, GD), lambda i: (i, 0)),
        compiler_params=pltpu.CompilerParams(
            dimension_semantics=("parallel",),
            vmem_limit_bytes=vmem_limit,
        ),
    )(idx2, w_blk)

    out = out_packed.reshape(Np, D)
    if Np != N:
        out = out[:N]
    return out.reshape(*orig_shape, D)


# speedup vs baseline: 2.0035x; 2.0035x over previous
import functools
import math

import jax
import jax.numpy as jnp
from jax import lax
from jax.experimental import pallas as pl
from jax.experimental.pallas import tpu as pltpu


def _cdiv(a, b):
    return -(-a // b)


def _gather_kernel(idxt_ref, w_ref, out_ref, *, num_categories, chunks):
    c = num_categories
    w = w_ref[...]
    idxt = jnp.clip(idxt_ref[...], 0, c - 1)
    iota_c = lax.broadcasted_iota(jnp.int32, (128, c), 1)
    for s in range(chunks):
        onehot = (iota_c == idxt[:, s:s + 1]).astype(jnp.bfloat16)
        out_ref[pl.ds(128 * s, 128), :] = jax.lax.dot_general(
            onehot, w,
            dimension_numbers=(((1,), (0,)), ((), ())),
            preferred_element_type=jnp.float32,
        )


def kernel(category_indices, weight):
    C, D = weight.shape
    orig_shape = category_indices.shape

    idx = category_indices.reshape(-1).astype(jnp.int32)
    N = idx.shape[0]

    chunks = 128
    tile = 128 * chunks
    n_tiles = _cdiv(N, tile)
    Npad = n_tiles * tile
    if Npad != N:
        idx = jnp.pad(idx, (0, Npad - N))

    idx_t = (idx.reshape(n_tiles, chunks, 128)
                .transpose(0, 2, 1)
                .reshape(n_tiles * 128, chunks))

    w16 = weight.astype(jnp.bfloat16)

    out = pl.pallas_call(
        functools.partial(_gather_kernel, num_categories=C, chunks=chunks),
        out_shape=jax.ShapeDtypeStruct((N, D), jnp.float32),
        grid=(n_tiles,),
        in_specs=[
            pl.BlockSpec((128, chunks), lambda i: (i, 0)),
            pl.BlockSpec((C, D), lambda i: (0, 0)),
        ],
        out_specs=pl.BlockSpec((tile, D), lambda i: (i, 0)),
        compiler_params=pltpu.CompilerParams(
            dimension_semantics=("parallel",),
        ),
    )(idx_t, w16)

    return out.reshape(*orig_shape, D)
